# Initial kernel scaffold; baseline (speedup 1.0000x reference)
import jax
import jax.numpy as jnp
from jax.experimental import pallas as pl


def kernel(x, edge_index, edge_weight, W1, b1, W2, b2, W3, b3, W4, b4, Ws1, bs1, Ws2, bs2, Wf, bf, g1, be1, g2, be2, g3, be3, g4, be4):
    raise NotImplementedError("write your pallas kernel here")


# speedup vs baseline: 10.7031x; 10.7031x over previous
import functools

import jax
import jax.numpy as jnp
import numpy as np
from jax import lax
from jax.experimental import pallas as pl
from jax.experimental.pallas import tpu as pltpu
from jax.experimental.pallas import tpu_sc as plsc

N = 10000
NPAD = 10240
E = 320000
NSUB = 16
NW = 2 * NSUB
CHUNK = 128
CPW = 79
EPAD = NW * CPW * CHUNK
RPS = NPAD // NSUB
BLK = 512
BN_C = float(1.0 / np.sqrt(1.0 + 1e-5))


def _sc_mesh():
    return plsc.VectorSubcoreMesh(core_axis_name="c", subcore_axis_name="s")


@functools.partial(
    pl.kernel,
    out_type=jax.ShapeDtypeStruct((2, NPAD), jnp.float32),
    mesh=_sc_mesh(),
    scratch_types=[
        pltpu.VMEM((CPW, CHUNK), jnp.int32),
        pltpu.VMEM((CPW, CHUNK), jnp.float32),
        pltpu.VMEM_SHARED((NPAD,), jnp.float32),
    ],
)
def _deg_kernel(col_hbm, w_hbm, zrow_hbm, out_hbm, col_v, w_v, acc):
    c = lax.axis_index("c")
    s = lax.axis_index("s")
    wid = c * NSUB + s
    r0 = s * RPS
    pltpu.sync_copy(zrow_hbm, acc.at[pl.ds(r0, RPS)])
    plsc.subcore_barrier()
    pltpu.sync_copy(col_hbm.at[wid], col_v)
    pltpu.sync_copy(w_hbm.at[wid], w_v)

    def step(t, carry):
        pltpu.sync_copy(w_v.at[t], acc.at[col_v.at[t]], add=True)
        return carry

    lax.fori_loop(0, CPW, step, 0)
    plsc.subcore_barrier()
    pltpu.sync_copy(acc.at[pl.ds(r0, RPS)], out_hbm.at[c, pl.ds(r0, RPS)])


def _make_agg(H):
    kh = H // 16

    @functools.partial(
        pl.kernel,
        out_type=jax.ShapeDtypeStruct((2, NPAD, H), jnp.float32),
        mesh=_sc_mesh(),
        scratch_types=[
            pltpu.VMEM((CPW, CHUNK), jnp.int32),
            pltpu.VMEM((CPW, CHUNK), jnp.int32),
            pltpu.VMEM((CPW, CHUNK), jnp.float32),
            pltpu.VMEM((CHUNK, H), jnp.float32),
            pltpu.VMEM_SHARED((NPAD, H), jnp.float32),
            pltpu.SemaphoreType.DMA,
        ],
        compiler_params=pltpu.CompilerParams(use_tc_tiling_on_sc=False),
    )
    def agg(hp_hbm, row_hbm, col_hbm, w_hbm, zrow_hbm, out_hbm,
            row_v, col_v, w_v, buf, acc, sem):
        c = lax.axis_index("c")
        s = lax.axis_index("s")
        wid = c * NSUB + s
        r0 = s * RPS
        pltpu.sync_copy(zrow_hbm, acc.at[pl.ds(r0, RPS)])
        plsc.subcore_barrier()
        pltpu.sync_copy(row_hbm.at[wid], row_v)
        pltpu.sync_copy(col_hbm.at[wid], col_v)
        pltpu.sync_copy(w_hbm.at[wid], w_v)

        def step(t, carry):
            pltpu.async_copy(hp_hbm.at[row_v.at[t]], buf, sem).wait()
            for j16 in range(CHUNK // 16):
                wv = w_v[t, pl.ds(j16 * 16, 16)]
                for jj in range(16):
                    j = j16 * 16 + jj
                    wj = wv[jj]
                    for k in range(kh):
                        sl = pl.ds(k * 16, 16)
                        buf[j, sl] = buf[j, sl] * wj
            pltpu.sync_copy(buf, acc.at[col_v.at[t]], add=True)
            return carry

        lax.fori_loop(0, CPW, step, 0)
        plsc.subcore_barrier()
        pltpu.sync_copy(acc.at[pl.ds(r0, RPS)], out_hbm.at[c, pl.ds(r0, RPS)])

    return agg


_agg128 = _make_agg(128)
_agg64 = _make_agg(64)
_agg32 = _make_agg(32)


def _dinv_of(deg_ref):
    d = deg_ref[0] + deg_ref[1] + 1.0
    return lax.rsqrt(d)


def _tc1_body(deg_ref, x_ref, w1_ref, ws1_ref, bs1_ref, hp1_ref, s1_ref):
    dinv = _dinv_of(deg_ref)
    x = x_ref[...]
    hp1_ref[...] = jnp.dot(x, w1_ref[...], preferred_element_type=jnp.float32) * dinv
    s1_ref[...] = jnp.dot(x, ws1_ref[...], preferred_element_type=jnp.float32) + bs1_ref[...]


def _tc2_body(deg_ref, agg_ref, hp1_ref, b1_ref, g1_ref, be1_ref, s1_ref,
              w2_ref, ws2_ref, bs2_ref, idn2_ref, hp2_ref, sk3_ref):
    dinv = _dinv_of(deg_ref)
    conv = dinv * (agg_ref[0] + agg_ref[1] + hp1_ref[...]) + b1_ref[...]
    z = conv * (g1_ref[...] * BN_C) + be1_ref[...] + s1_ref[...]
    idn2 = jnp.maximum(z, 0.0)
    idn2_ref[...] = idn2
    hp2_ref[...] = jnp.dot(idn2, w2_ref[...], preferred_element_type=jnp.float32) * dinv
    sk3_ref[...] = jnp.dot(idn2, ws2_ref[...], preferred_element_type=jnp.float32) + bs2_ref[...]


def _tc3_body(deg_ref, agg_ref, hp2_ref, b2_ref, g2_ref, be2_ref, idn2_ref,
              w3_ref, hp3_ref):
    dinv = _dinv_of(deg_ref)
    conv = dinv * (agg_ref[0] + agg_ref[1] + hp2_ref[...]) + b2_ref[...]
    h2 = jnp.maximum(conv * (g2_ref[...] * BN_C) + be2_ref[...] + idn2_ref[...], 0.0)
    hp3_ref[...] = jnp.dot(h2, w3_ref[...], preferred_element_type=jnp.float32) * dinv


def _tc4_body(deg_ref, agg_ref, hp3_ref, b3_ref, g3_ref, be3_ref, sk3_ref,
              w4_ref, hp4_ref):
    dinv = _dinv_of(deg_ref)
    conv = dinv * (agg_ref[0] + agg_ref[1] + hp3_ref[...]) + b3_ref[...]
    h3 = jnp.maximum(conv * (g3_ref[...] * BN_C) + be3_ref[...] + sk3_ref[...], 0.0)
    hp4_ref[...] = jnp.dot(h3, w4_ref[...], preferred_element_type=jnp.float32) * dinv


def _tc5_body(deg_ref, agg_ref, hp4_ref, b4_ref, g4_ref, be4_ref, wf_ref,
              bf_ref, out_ref):
    dinv = _dinv_of(deg_ref)
    conv = dinv * (agg_ref[0] + agg_ref[1] + hp4_ref[...]) + b4_ref[...]
    h4 = jnp.maximum(conv * (g4_ref[...] * BN_C) + be4_ref[...], 0.0)
    out_ref[...] = jnp.dot(h4, wf_ref[...], preferred_element_type=jnp.float32) + bf_ref[...]


def _row_spec(h):
    return pl.BlockSpec((BLK, h), lambda i: (i, 0))


def _full_spec(shape):
    nd = len(shape)
    return pl.BlockSpec(shape, lambda i: (0,) * nd)


_DEG_SPEC = pl.BlockSpec((2, BLK, 1), lambda i: (0, i, 0))


def _agg_spec(h):
    return pl.BlockSpec((2, BLK, h), lambda i: (0, i, 0))


def _tc_call(body, in_specs, out_specs, out_shapes):
    return pl.pallas_call(
        body,
        grid=(NPAD // BLK,),
        in_specs=in_specs,
        out_specs=out_specs,
        out_shape=out_shapes,
    )


def kernel(x, edge_index, edge_weight, W1, b1, W2, b2, W3, b3, W4, b4,
           Ws1, bs1, Ws2, bs2, Wf, bf, g1, be1, g2, be2, g3, be3, g4, be4):
    f32 = jnp.float32
    pad_e = EPAD - E
    rowp = jnp.concatenate([edge_index[0], jnp.zeros((pad_e,), jnp.int32)]
                           ).reshape(NW, CPW, CHUNK)
    colp = jnp.concatenate([edge_index[1], jnp.zeros((pad_e,), jnp.int32)]
                           ).reshape(NW, CPW, CHUNK)
    wp = jnp.concatenate([edge_weight, jnp.zeros((pad_e,), f32)]
                         ).reshape(NW, CPW, CHUNK)
    xp = jnp.pad(x, ((0, NPAD - N), (0, 0)))
    z1 = jnp.zeros((RPS,), f32)
    z128 = jnp.zeros((RPS, 128), f32)
    z64 = jnp.zeros((RPS, 64), f32)
    z32 = jnp.zeros((RPS, 32), f32)
    r = lambda v: v.reshape(1, -1)

    deg2 = _deg_kernel(colp, wp, z1).reshape(2, NPAD, 1)

    hp1, s1 = _tc_call(
        _tc1_body,
        [_DEG_SPEC, _row_spec(128), _full_spec((128, 128)),
         _full_spec((128, 128)), _full_spec((1, 128))],
        [_row_spec(128), _row_spec(128)],
        [jax.ShapeDtypeStruct((NPAD, 128), f32)] * 2,
    )(deg2, xp, W1, Ws1, r(bs1))

    agg1 = _agg128(hp1, rowp, colp, wp, z128)

    idn2, hp2, sk3 = _tc_call(
        _tc2_body,
        [_DEG_SPEC, _agg_spec(128), _row_spec(128), _full_spec((1, 128)),
         _full_spec((1, 128)), _full_spec((1, 128)), _row_spec(128),
         _full_spec((128, 128)), _full_spec((128, 64)), _full_spec((1, 64))],
        [_row_spec(128), _row_spec(128), _row_spec(64)],
        [jax.ShapeDtypeStruct((NPAD, 128), f32),
         jax.ShapeDtypeStruct((NPAD, 128), f32),
         jax.ShapeDtypeStruct((NPAD, 64), f32)],
    )(deg2, agg1, hp1, r(b1), r(g1), r(be1), s1, W2, Ws2, r(bs2))

    agg2 = _agg128(hp2, rowp, colp, wp, z128)

    hp3 = _tc_call(
        _tc3_body,
        [_DEG_SPEC, _agg_spec(128), _row_spec(128), _full_spec((1, 128)),
         _full_spec((1, 128)), _full_spec((1, 128)), _row_spec(128),
         _full_spec((128, 64))],
        [_row_spec(64)],
        [jax.ShapeDtypeStruct((NPAD, 64), f32)],
    )(deg2, agg2, hp2, r(b2), r(g2), r(be2), idn2, W3)[0]

    agg3 = _agg64(hp3, rowp, colp, wp, z64)

    hp4 = _tc_call(
        _tc4_body,
        [_DEG_SPEC, _agg_spec(64), _row_spec(64), _full_spec((1, 64)),
         _full_spec((1, 64)), _full_spec((1, 64)), _row_spec(64),
         _full_spec((64, 32))],
        [_row_spec(32)],
        [jax.ShapeDtypeStruct((NPAD, 32), f32)],
    )(deg2, agg3, hp3, r(b3), r(g3), r(be3), sk3, W4)[0]

    agg4 = _agg32(hp4, rowp, colp, wp, z32)

    outp = _tc_call(
        _tc5_body,
        [_DEG_SPEC, _agg_spec(32), _row_spec(32), _full_spec((1, 32)),
         _full_spec((1, 32)), _full_spec((1, 32)), _full_spec((32, 2)),
         _full_spec((1, 2))],
        [_row_spec(2)],
        [jax.ShapeDtypeStruct((NPAD, 2), f32)],
    )(deg2, agg4, hp4, r(b4), r(g4), r(be4), Wf, r(bf))[0]

    return outp[:N]


# speedup vs baseline: 12.7283x; 1.1892x over previous
import functools

import jax
import jax.numpy as jnp
import numpy as np
from jax import lax
from jax.experimental import pallas as pl
from jax.experimental.pallas import tpu as pltpu
from jax.experimental.pallas import tpu_sc as plsc

N = 10000
NPAD = 10240
E = 320000
NSUB = 16
CHUNK = 128
CPT = 160
PAIRS = CPT // 2
HCPT = CPT // 2
EPAD = NSUB * CPT * CHUNK
RPS = NPAD // NSUB
BLK = 512
BN_C = float(1.0 / np.sqrt(1.0 + 1e-5))


def _sc_mesh():
    return plsc.VectorSubcoreMesh(core_axis_name="c", subcore_axis_name="s")


@functools.partial(
    pl.kernel,
    out_type=jax.ShapeDtypeStruct((2, NPAD), jnp.float32),
    mesh=_sc_mesh(),
    scratch_types=[
        pltpu.VMEM((HCPT, CHUNK), jnp.int32),
        pltpu.VMEM((HCPT, CHUNK), jnp.float32),
        pltpu.VMEM_SHARED((NPAD,), jnp.float32),
    ],
    compiler_params=pltpu.CompilerParams(use_tc_tiling_on_sc=False),
)
def _deg_kernel(col_hbm, w_hbm, zrow_hbm, out_hbm, col_v, w_v, acc):
    c = lax.axis_index("c")
    s = lax.axis_index("s")
    r0 = s * RPS
    pltpu.sync_copy(zrow_hbm, acc.at[pl.ds(r0, RPS)])
    plsc.subcore_barrier()
    pltpu.sync_copy(col_hbm.at[s, pl.ds(c * HCPT, HCPT)], col_v)
    pltpu.sync_copy(w_hbm.at[s, pl.ds(c * HCPT, HCPT)], w_v)

    def step(t, carry):
        pltpu.sync_copy(w_v.at[t], acc.at[col_v.at[t]], add=True)
        return carry

    lax.fori_loop(0, HCPT, step, 0)
    plsc.subcore_barrier()
    pltpu.sync_copy(acc.at[pl.ds(r0, RPS)], out_hbm.at[c, pl.ds(r0, RPS)])


def _make_agg(H):
    HH = H // 2
    kh = HH // 16

    @functools.partial(
        pl.kernel,
        out_type=jax.ShapeDtypeStruct((NPAD, H), jnp.float32),
        mesh=_sc_mesh(),
        scratch_types=[
            pltpu.VMEM((CPT, CHUNK), jnp.int32),
            pltpu.VMEM((CPT, CHUNK), jnp.int32),
            pltpu.VMEM((CPT, CHUNK), jnp.float32),
            pltpu.VMEM((CHUNK, HH), jnp.float32),
            pltpu.VMEM((CHUNK, HH), jnp.float32),
            pltpu.VMEM_SHARED((NPAD, HH), jnp.float32),
            pltpu.SemaphoreType.DMA,
            pltpu.SemaphoreType.DMA,
        ],
        compiler_params=pltpu.CompilerParams(use_tc_tiling_on_sc=False),
    )
    def agg(hp_hbm, row_hbm, col_hbm, w_hbm, zrow_hbm, out_hbm,
            row_v, col_v, w_v, buf_a, buf_b, acc, sem_a, sem_b):
        c = lax.axis_index("c")
        s = lax.axis_index("s")
        r0 = s * RPS
        pltpu.sync_copy(zrow_hbm, acc.at[pl.ds(r0, RPS)])
        plsc.subcore_barrier()
        pltpu.sync_copy(row_hbm.at[s], row_v)
        pltpu.sync_copy(col_hbm.at[s], col_v)
        pltpu.sync_copy(w_hbm.at[s], w_v)

        def gather(t, buf, sem):
            return pltpu.make_async_copy(hp_hbm.at[c].at[row_v.at[t]], buf, sem)

        def scale_scatter(t, buf):
            for j16 in range(CHUNK // 16):
                wv = w_v[t, pl.ds(j16 * 16, 16)]
                for jj in range(16):
                    j = j16 * 16 + jj
                    wj = wv[jj]
                    for k in range(kh):
                        sl = pl.ds(k * 16, 16)
                        buf[j, sl] = buf[j, sl] * wj
            pltpu.sync_copy(buf, acc.at[col_v.at[t]], add=True)

        gather(0, buf_a, sem_a).start()
        gather(1, buf_b, sem_b).start()

        def step(p, carry):
            ta = 2 * p
            gather(ta, buf_a, sem_a).wait()
            scale_scatter(ta, buf_a)

            @pl.when(p < PAIRS - 1)
            def _():
                gather(ta + 2, buf_a, sem_a).start()

            gather(ta + 1, buf_b, sem_b).wait()
            scale_scatter(ta + 1, buf_b)

            @pl.when(p < PAIRS - 1)
            def _():
                gather(ta + 3, buf_b, sem_b).start()

            return carry

        lax.fori_loop(0, PAIRS, step, 0)
        plsc.subcore_barrier()
        pltpu.sync_copy(acc.at[pl.ds(r0, RPS)],
                        out_hbm.at[pl.ds(r0, RPS), pl.ds(c * HH, HH)])

    return agg


_agg128 = _make_agg(128)
_agg64 = _make_agg(64)
_agg32 = _make_agg(32)


def _dinv_of(deg_ref):
    d = deg_ref[0] + deg_ref[1] + 1.0
    return lax.rsqrt(d)


def _split(hp_ref, h):
    hh = h.shape[-1] // 2
    hp_ref[0] = h[:, :hh]
    hp_ref[1] = h[:, hh:]


def _cat(hp_ref):
    return jnp.concatenate([hp_ref[0], hp_ref[1]], axis=-1)


def _tc1_body(deg_ref, x_ref, w1_ref, ws1_ref, bs1_ref, hp1_ref, s1_ref):
    dinv = _dinv_of(deg_ref)
    x = x_ref[...]
    _split(hp1_ref, jnp.dot(x, w1_ref[...], preferred_element_type=jnp.float32) * dinv)
    s1_ref[...] = jnp.dot(x, ws1_ref[...], preferred_element_type=jnp.float32) + bs1_ref[...]


def _tc2_body(deg_ref, agg_ref, hp1_ref, b1_ref, g1_ref, be1_ref, s1_ref,
              w2_ref, ws2_ref, bs2_ref, idn2_ref, hp2_ref, sk3_ref):
    dinv = _dinv_of(deg_ref)
    conv = dinv * (agg_ref[...] + _cat(hp1_ref)) + b1_ref[...]
    z = conv * (g1_ref[...] * BN_C) + be1_ref[...] + s1_ref[...]
    idn2 = jnp.maximum(z, 0.0)
    idn2_ref[...] = idn2
    _split(hp2_ref, jnp.dot(idn2, w2_ref[...], preferred_element_type=jnp.float32) * dinv)
    sk3_ref[...] = jnp.dot(idn2, ws2_ref[...], preferred_element_type=jnp.float32) + bs2_ref[...]


def _tc3_body(deg_ref, agg_ref, hp2_ref, b2_ref, g2_ref, be2_ref, idn2_ref,
              w3_ref, hp3_ref):
    dinv = _dinv_of(deg_ref)
    conv = dinv * (agg_ref[...] + _cat(hp2_ref)) + b2_ref[...]
    h2 = jnp.maximum(conv * (g2_ref[...] * BN_C) + be2_ref[...] + idn2_ref[...], 0.0)
    _split(hp3_ref, jnp.dot(h2, w3_ref[...], preferred_element_type=jnp.float32) * dinv)


def _tc4_body(deg_ref, agg_ref, hp3_ref, b3_ref, g3_ref, be3_ref, sk3_ref,
              w4_ref, hp4_ref):
    dinv = _dinv_of(deg_ref)
    conv = dinv * (agg_ref[...] + _cat(hp3_ref)) + b3_ref[...]
    h3 = jnp.maximum(conv * (g3_ref[...] * BN_C) + be3_ref[...] + sk3_ref[...], 0.0)
    _split(hp4_ref, jnp.dot(h3, w4_ref[...], preferred_element_type=jnp.float32) * dinv)


def _tc5_body(deg_ref, agg_ref, hp4_ref, b4_ref, g4_ref, be4_ref, wf_ref,
              bf_ref, out_ref):
    dinv = _dinv_of(deg_ref)
    conv = dinv * (agg_ref[...] + _cat(hp4_ref)) + b4_ref[...]
    h4 = jnp.maximum(conv * (g4_ref[...] * BN_C) + be4_ref[...], 0.0)
    out_ref[...] = jnp.dot(h4, wf_ref[...], preferred_element_type=jnp.float32) + bf_ref[...]


def _row_spec(h):
    return pl.BlockSpec((BLK, h), lambda i: (i, 0))


def _full_spec(shape):
    nd = len(shape)
    return pl.BlockSpec(shape, lambda i: (0,) * nd)


_DEG_SPEC = pl.BlockSpec((2, BLK, 1), lambda i: (0, i, 0))


def _hp_spec(hh):
    return pl.BlockSpec((2, BLK, hh), lambda i: (0, i, 0))


def _tc_call(body, in_specs, out_specs, out_shapes):
    return pl.pallas_call(
        body,
        grid=(NPAD // BLK,),
        in_specs=in_specs,
        out_specs=out_specs,
        out_shape=out_shapes,
    )


def kernel(x, edge_index, edge_weight, W1, b1, W2, b2, W3, b3, W4, b4,
           Ws1, bs1, Ws2, bs2, Wf, bf, g1, be1, g2, be2, g3, be3, g4, be4):
    f32 = jnp.float32
    pad_e = EPAD - E
    rowp = jnp.concatenate([edge_index[0], jnp.zeros((pad_e,), jnp.int32)]
                           ).reshape(NSUB, CPT, CHUNK)
    colp = jnp.concatenate([edge_index[1], jnp.zeros((pad_e,), jnp.int32)]
                           ).reshape(NSUB, CPT, CHUNK)
    wp = jnp.concatenate([edge_weight, jnp.zeros((pad_e,), f32)]
                         ).reshape(NSUB, CPT, CHUNK)
    xp = jnp.pad(x, ((0, NPAD - N), (0, 0)))
    z1 = jnp.zeros((RPS,), f32)
    z64 = jnp.zeros((RPS, 64), f32)
    z32 = jnp.zeros((RPS, 32), f32)
    z16 = jnp.zeros((RPS, 16), f32)
    r = lambda v: v.reshape(1, -1)

    deg2 = _deg_kernel(colp, wp, z1).reshape(2, NPAD, 1)

    hp1, s1 = _tc_call(
        _tc1_body,
        [_DEG_SPEC, _row_spec(128), _full_spec((128, 128)),
         _full_spec((128, 128)), _full_spec((1, 128))],
        [_hp_spec(64), _row_spec(128)],
        [jax.ShapeDtypeStruct((2, NPAD, 64), f32),
         jax.ShapeDtypeStruct((NPAD, 128), f32)],
    )(deg2, xp, W1, Ws1, r(bs1))

    agg1 = _agg128(hp1, rowp, colp, wp, z64)

    idn2, hp2, sk3 = _tc_call(
        _tc2_body,
        [_DEG_SPEC, _row_spec(128), _hp_spec(64), _full_spec((1, 128)),
         _full_spec((1, 128)), _full_spec((1, 128)), _row_spec(128),
         _full_spec((128, 128)), _full_spec((128, 64)), _full_spec((1, 64))],
        [_row_spec(128), _hp_spec(64), _row_spec(64)],
        [jax.ShapeDtypeStruct((NPAD, 128), f32),
         jax.ShapeDtypeStruct((2, NPAD, 64), f32),
         jax.ShapeDtypeStruct((NPAD, 64), f32)],
    )(deg2, agg1, hp1, r(b1), r(g1), r(be1), s1, W2, Ws2, r(bs2))

    agg2 = _agg128(hp2, rowp, colp, wp, z64)

    hp3 = _tc_call(
        _tc3_body,
        [_DEG_SPEC, _row_spec(128), _hp_spec(64), _full_spec((1, 128)),
         _full_spec((1, 128)), _full_spec((1, 128)), _row_spec(128),
         _full_spec((128, 64))],
        [_hp_spec(32)],
        [jax.ShapeDtypeStruct((2, NPAD, 32), f32)],
    )(deg2, agg2, hp2, r(b2), r(g2), r(be2), idn2, W3)[0]

    agg3 = _agg64(hp3, rowp, colp, wp, z32)

    hp4 = _tc_call(
        _tc4_body,
        [_DEG_SPEC, _row_spec(64), _hp_spec(32), _full_spec((1, 64)),
         _full_spec((1, 64)), _full_spec((1, 64)), _row_spec(64),
         _full_spec((64, 32))],
        [_hp_spec(16)],
        [jax.ShapeDtypeStruct((2, NPAD, 16), f32)],
    )(deg2, agg3, hp3, r(b3), r(g3), r(be3), sk3, W4)[0]

    agg4 = _agg32(hp4, rowp, colp, wp, z16)

    outp = _tc_call(
        _tc5_body,
        [_DEG_SPEC, _row_spec(32), _hp_spec(16), _full_spec((1, 32)),
         _full_spec((1, 32)), _full_spec((1, 32)), _full_spec((32, 2)),
         _full_spec((1, 2))],
        [_row_spec(2)],
        [jax.ShapeDtypeStruct((NPAD, 2), f32)],
    )(deg2, agg4, hp4, r(b4), r(g4), r(be4), Wf, r(bf))[0]

    return outp[:N]


# speedup vs baseline: 13.8556x; 1.0886x over previous
import functools

import jax
import jax.numpy as jnp
import numpy as np
from jax import lax
from jax.experimental import pallas as pl
from jax.experimental.pallas import tpu as pltpu
from jax.experimental.pallas import tpu_sc as plsc

N = 10000
NPAD = 10240
E = 320000
NSUB = 16
CHUNK = 96
CPT = 212
HCPT = CPT // 2
EPAD = NSUB * CPT * CHUNK
RPS = NPAD // NSUB
BLK = 512
BN_C = float(1.0 / np.sqrt(1.0 + 1e-5))


def _sc_mesh():
    return plsc.VectorSubcoreMesh(core_axis_name="c", subcore_axis_name="s")


@functools.partial(
    pl.kernel,
    out_type=jax.ShapeDtypeStruct((2, NPAD), jnp.float32),
    mesh=_sc_mesh(),
    scratch_types=[
        pltpu.VMEM((HCPT, CHUNK), jnp.int32),
        pltpu.VMEM((HCPT, CHUNK), jnp.float32),
        pltpu.VMEM_SHARED((NPAD,), jnp.float32),
    ],
    compiler_params=pltpu.CompilerParams(use_tc_tiling_on_sc=False),
)
def _deg_kernel(col_hbm, w_hbm, zrow_hbm, out_hbm, col_v, w_v, acc):
    c = lax.axis_index("c")
    s = lax.axis_index("s")
    r0 = s * RPS
    pltpu.sync_copy(zrow_hbm, acc.at[pl.ds(r0, RPS)])
    plsc.subcore_barrier()
    pltpu.sync_copy(col_hbm.at[s, pl.ds(c * HCPT, HCPT)], col_v)
    pltpu.sync_copy(w_hbm.at[s, pl.ds(c * HCPT, HCPT)], w_v)

    def step(t, carry):
        pltpu.sync_copy(w_v.at[t], acc.at[col_v.at[t]], add=True)
        return carry

    lax.fori_loop(0, HCPT, step, 0)
    plsc.subcore_barrier()
    pltpu.sync_copy(acc.at[pl.ds(r0, RPS)], out_hbm.at[c, pl.ds(r0, RPS)])


def _make_agg(H):
    HH = H // 2
    kh = HH // 16

    @functools.partial(
        pl.kernel,
        out_type=jax.ShapeDtypeStruct((NPAD, H), jnp.float32),
        mesh=_sc_mesh(),
        scratch_types=[
            pltpu.VMEM((CPT, CHUNK), jnp.int32),
            pltpu.VMEM((CPT, CHUNK), jnp.int32),
            pltpu.VMEM((CPT, CHUNK), jnp.float32),
            pltpu.VMEM((CHUNK, HH), jnp.float32),
            pltpu.VMEM((CHUNK, HH), jnp.float32),
            pltpu.VMEM((CHUNK, HH), jnp.float32),
            pltpu.VMEM((CHUNK, HH), jnp.float32),
            pltpu.VMEM_SHARED((NPAD, HH), jnp.float32),
            pltpu.SemaphoreType.DMA,
            pltpu.SemaphoreType.DMA,
            pltpu.SemaphoreType.DMA,
            pltpu.SemaphoreType.DMA,
            pltpu.SemaphoreType.DMA,
            pltpu.SemaphoreType.DMA,
            pltpu.SemaphoreType.DMA,
            pltpu.SemaphoreType.DMA,
        ],
        compiler_params=pltpu.CompilerParams(use_tc_tiling_on_sc=False),
    )
    def agg(hp_hbm, row_hbm, col_hbm, w_hbm, zrow_hbm, out_hbm,
            row_v, col_v, w_v, b0, b1, b2, b3, acc,
            g0, g1, g2, g3, s0, s1, s2, s3):
        c = lax.axis_index("c")
        s = lax.axis_index("s")
        r0 = s * RPS
        pltpu.sync_copy(zrow_hbm, acc.at[pl.ds(r0, RPS)])
        plsc.subcore_barrier()
        pltpu.sync_copy(row_hbm.at[s], row_v)
        pltpu.sync_copy(col_hbm.at[s], col_v)
        pltpu.sync_copy(w_hbm.at[s], w_v)

        bufs = (b0, b1, b2, b3)
        gsems = (g0, g1, g2, g3)
        ssems = (s0, s1, s2, s3)

        def gather_start(t, i):
            pltpu.async_copy(hp_hbm.at[c].at[row_v.at[t]], bufs[i], gsems[i])

        def gather_wait(t, i):
            pltpu.make_async_copy(hp_hbm.at[c].at[row_v.at[t]], bufs[i],
                                  gsems[i]).wait()

        def scatter_start(t, i):
            pltpu.async_copy(bufs[i], acc.at[col_v.at[t]], ssems[i], add=True)

        def scatter_wait(t, i):
            pltpu.make_async_copy(bufs[i], acc.at[col_v.at[t]], ssems[i]).wait()

        def scale(t, buf):
            for j16 in range(CHUNK // 16):
                wv = w_v[t, pl.ds(j16 * 16, 16)]
                for jj in range(16):
                    j = j16 * 16 + jj
                    wj = wv[jj]
                    for k in range(kh):
                        sl = pl.ds(k * 16, 16)
                        buf[j, sl] = buf[j, sl] * wj

        gather_start(0, 0)
        gather_start(1, 1)

        def step(p, carry):
            for b in range(4):
                t = 4 * p + b
                rb = (b + 2) % 4
                gather_wait(t, b)
                scale(t, bufs[b])
                scatter_start(t, b)

                @pl.when(t >= 2)
                def _():
                    scatter_wait(t - 2, rb)

                @pl.when(t + 2 < CPT)
                def _():
                    gather_start(t + 2, rb)

            return carry

        lax.fori_loop(0, CPT // 4, step, 0)
        scatter_wait(CPT - 2, (CPT - 2) % 4)
        scatter_wait(CPT - 1, (CPT - 1) % 4)
        plsc.subcore_barrier()
        pltpu.sync_copy(acc.at[pl.ds(r0, RPS)],
                        out_hbm.at[pl.ds(r0, RPS), pl.ds(c * HH, HH)])

    return agg


_agg128 = _make_agg(128)
_agg64 = _make_agg(64)
_agg32 = _make_agg(32)


def _dinv_of(deg_ref):
    d = deg_ref[0] + deg_ref[1] + 1.0
    return lax.rsqrt(d)


def _split(hp_ref, h):
    hh = h.shape[-1] // 2
    hp_ref[0] = h[:, :hh]
    hp_ref[1] = h[:, hh:]


def _cat(hp_ref):
    return jnp.concatenate([hp_ref[0], hp_ref[1]], axis=-1)


def _tc1_body(deg_ref, x_ref, w1_ref, ws1_ref, bs1_ref, hp1_ref, s1_ref):
    dinv = _dinv_of(deg_ref)
    x = x_ref[...]
    _split(hp1_ref, jnp.dot(x, w1_ref[...], preferred_element_type=jnp.float32) * dinv)
    s1_ref[...] = jnp.dot(x, ws1_ref[...], preferred_element_type=jnp.float32) + bs1_ref[...]


def _tc2_body(deg_ref, agg_ref, hp1_ref, b1_ref, g1_ref, be1_ref, s1_ref,
              w2_ref, ws2_ref, bs2_ref, idn2_ref, hp2_ref, sk3_ref):
    dinv = _dinv_of(deg_ref)
    conv = dinv * (agg_ref[...] + _cat(hp1_ref)) + b1_ref[...]
    z = conv * (g1_ref[...] * BN_C) + be1_ref[...] + s1_ref[...]
    idn2 = jnp.maximum(z, 0.0)
    idn2_ref[...] = idn2
    _split(hp2_ref, jnp.dot(idn2, w2_ref[...], preferred_element_type=jnp.float32) * dinv)
    sk3_ref[...] = jnp.dot(idn2, ws2_ref[...], preferred_element_type=jnp.float32) + bs2_ref[...]


def _tc3_body(deg_ref, agg_ref, hp2_ref, b2_ref, g2_ref, be2_ref, idn2_ref,
              w3_ref, hp3_ref):
    dinv = _dinv_of(deg_ref)
    conv = dinv * (agg_ref[...] + _cat(hp2_ref)) + b2_ref[...]
    h2 = jnp.maximum(conv * (g2_ref[...] * BN_C) + be2_ref[...] + idn2_ref[...], 0.0)
    _split(hp3_ref, jnp.dot(h2, w3_ref[...], preferred_element_type=jnp.float32) * dinv)


def _tc4_body(deg_ref, agg_ref, hp3_ref, b3_ref, g3_ref, be3_ref, sk3_ref,
              w4_ref, hp4_ref):
    dinv = _dinv_of(deg_ref)
    conv = dinv * (agg_ref[...] + _cat(hp3_ref)) + b3_ref[...]
    h3 = jnp.maximum(conv * (g3_ref[...] * BN_C) + be3_ref[...] + sk3_ref[...], 0.0)
    _split(hp4_ref, jnp.dot(h3, w4_ref[...], preferred_element_type=jnp.float32) * dinv)


def _tc5_body(deg_ref, agg_ref, hp4_ref, b4_ref, g4_ref, be4_ref, wf_ref,
              bf_ref, out_ref):
    dinv = _dinv_of(deg_ref)
    conv = dinv * (agg_ref[...] + _cat(hp4_ref)) + b4_ref[...]
    h4 = jnp.maximum(conv * (g4_ref[...] * BN_C) + be4_ref[...], 0.0)
    out_ref[...] = jnp.dot(h4, wf_ref[...], preferred_element_type=jnp.float32) + bf_ref[...]


def _row_spec(h):
    return pl.BlockSpec((BLK, h), lambda i: (i, 0))


def _full_spec(shape):
    nd = len(shape)
    return pl.BlockSpec(shape, lambda i: (0,) * nd)


_DEG_SPEC = pl.BlockSpec((2, BLK, 1), lambda i: (0, i, 0))


def _hp_spec(hh):
    return pl.BlockSpec((2, BLK, hh), lambda i: (0, i, 0))


def _tc_call(body, in_specs, out_specs, out_shapes):
    return pl.pallas_call(
        body,
        grid=(NPAD // BLK,),
        in_specs=in_specs,
        out_specs=out_specs,
        out_shape=out_shapes,
    )


def kernel(x, edge_index, edge_weight, W1, b1, W2, b2, W3, b3, W4, b4,
           Ws1, bs1, Ws2, bs2, Wf, bf, g1, be1, g2, be2, g3, be3, g4, be4):
    f32 = jnp.float32
    pad_e = EPAD - E
    rowp = jnp.concatenate([edge_index[0], jnp.zeros((pad_e,), jnp.int32)]
                           ).reshape(NSUB, CPT, CHUNK)
    colp = jnp.concatenate([edge_index[1], jnp.zeros((pad_e,), jnp.int32)]
                           ).reshape(NSUB, CPT, CHUNK)
    wp = jnp.concatenate([edge_weight, jnp.zeros((pad_e,), f32)]
                         ).reshape(NSUB, CPT, CHUNK)
    xp = jnp.pad(x, ((0, NPAD - N), (0, 0)))
    z1 = jnp.zeros((RPS,), f32)
    z64 = jnp.zeros((RPS, 64), f32)
    z32 = jnp.zeros((RPS, 32), f32)
    z16 = jnp.zeros((RPS, 16), f32)
    r = lambda v: v.reshape(1, -1)

    deg2 = _deg_kernel(colp, wp, z1).reshape(2, NPAD, 1)

    hp1, s1 = _tc_call(
        _tc1_body,
        [_DEG_SPEC, _row_spec(128), _full_spec((128, 128)),
         _full_spec((128, 128)), _full_spec((1, 128))],
        [_hp_spec(64), _row_spec(128)],
        [jax.ShapeDtypeStruct((2, NPAD, 64), f32),
         jax.ShapeDtypeStruct((NPAD, 128), f32)],
    )(deg2, xp, W1, Ws1, r(bs1))

    agg1 = _agg128(hp1, rowp, colp, wp, z64)

    idn2, hp2, sk3 = _tc_call(
        _tc2_body,
        [_DEG_SPEC, _row_spec(128), _hp_spec(64), _full_spec((1, 128)),
         _full_spec((1, 128)), _full_spec((1, 128)), _row_spec(128),
         _full_spec((128, 128)), _full_spec((128, 64)), _full_spec((1, 64))],
        [_row_spec(128), _hp_spec(64), _row_spec(64)],
        [jax.ShapeDtypeStruct((NPAD, 128), f32),
         jax.ShapeDtypeStruct((2, NPAD, 64), f32),
         jax.ShapeDtypeStruct((NPAD, 64), f32)],
    )(deg2, agg1, hp1, r(b1), r(g1), r(be1), s1, W2, Ws2, r(bs2))

    agg2 = _agg128(hp2, rowp, colp, wp, z64)

    hp3 = _tc_call(
        _tc3_body,
        [_DEG_SPEC, _row_spec(128), _hp_spec(64), _full_spec((1, 128)),
         _full_spec((1, 128)), _full_spec((1, 128)), _row_spec(128),
         _full_spec((128, 64))],
        [_hp_spec(32)],
        [jax.ShapeDtypeStruct((2, NPAD, 32), f32)],
    )(deg2, agg2, hp2, r(b2), r(g2), r(be2), idn2, W3)[0]

    agg3 = _agg64(hp3, rowp, colp, wp, z32)

    hp4 = _tc_call(
        _tc4_body,
        [_DEG_SPEC, _row_spec(64), _hp_spec(32), _full_spec((1, 64)),
         _full_spec((1, 64)), _full_spec((1, 64)), _row_spec(64),
         _full_spec((64, 32))],
        [_hp_spec(16)],
        [jax.ShapeDtypeStruct((2, NPAD, 16), f32)],
    )(deg2, agg3, hp3, r(b3), r(g3), r(be3), sk3, W4)[0]

    agg4 = _agg32(hp4, rowp, colp, wp, z16)

    outp = _tc_call(
        _tc5_body,
        [_DEG_SPEC, _row_spec(32), _hp_spec(16), _full_spec((1, 32)),
         _full_spec((1, 32)), _full_spec((1, 32)), _full_spec((32, 2)),
         _full_spec((1, 2))],
        [_row_spec(2)],
        [jax.ShapeDtypeStruct((NPAD, 2), f32)],
    )(deg2, agg4, hp4, r(b4), r(g4), r(be4), Wf, r(bf))[0]

    return outp[:N]
